# Initial kernel scaffold; baseline (speedup 1.0000x reference)
import jax
import jax.numpy as jnp
from jax.experimental import pallas as pl


def kernel(l_v, h, s_c, iterations, b, w):
    raise NotImplementedError("write your pallas kernel here")


# speedup vs baseline: 2.6712x; 2.6712x over previous
import functools

import jax
import jax.numpy as jnp
from jax.experimental import pallas as pl

_E_TILE = 256


def _row_prod(x):
    n = x.shape[-1]
    while n > 128:
        half = n // 2
        x = x[:, :half] * x[:, half:n]
        n = half
    while n > 1:
        half = n // 2
        x = x[:, :half] * x[:, half:n]
        n = half
    return x


def _bp_iter_kernel(c_ref, mu_ref, tot_ref, base_ref, sign_ref,
                    mu_out_ref, tot_out_ref):
    j = pl.program_id(0)
    c = c_ref[...]
    mask = c >= 0.0
    wv = jnp.where(mask, c, 0.0)
    contrib = mu_ref[...] * wv
    m = base_ref[...] + tot_ref[...] - contrib
    t = jnp.where(mask, jnp.tanh(m * 0.5), 1.0)
    is_zero = t == 0.0
    nzv = jnp.where(is_zero, 1.0, t)
    prod_nz = _row_prod(nzv)
    zero_cnt = jnp.sum(is_zero.astype(jnp.float32), axis=1, keepdims=True)
    full_div = prod_nz / nzv
    sel = (zero_cnt == 0.0) | ((zero_cnt == 1.0) & is_zero)
    excl = jnp.where(sel, full_div, 0.0)
    mu_new = sign_ref[...] * jnp.log((1.0 + excl) / (1.0 - excl))
    mu_out_ref[...] = mu_new
    part = jnp.sum(mu_new * wv, axis=0, keepdims=True)

    @pl.when(j == 0)
    def _():
        tot_out_ref[...] = jnp.zeros_like(tot_out_ref)

    tot_out_ref[...] += part


def _bp_iteration(c, base2d, sign2d, mu, tot):
    num_edges, num_nodes = c.shape
    t = _E_TILE
    grid = (num_edges // t,)
    return pl.pallas_call(
        _bp_iter_kernel,
        grid=grid,
        in_specs=[
            pl.BlockSpec((t, num_nodes), lambda j: (j, 0)),
            pl.BlockSpec((t, num_nodes), lambda j: (j, 0)),
            pl.BlockSpec((1, num_nodes), lambda j: (0, 0)),
            pl.BlockSpec((1, num_nodes), lambda j: (0, 0)),
            pl.BlockSpec((t, 1), lambda j: (j, 0)),
        ],
        out_specs=[
            pl.BlockSpec((t, num_nodes), lambda j: (j, 0)),
            pl.BlockSpec((1, num_nodes), lambda j: (0, 0)),
        ],
        out_shape=[
            jax.ShapeDtypeStruct((num_edges, num_nodes), jnp.float32),
            jax.ShapeDtypeStruct((1, num_nodes), jnp.float32),
        ],
    )(c, mu, tot, base2d, sign2d)


def kernel(l_v, h, s_c, iterations, b, w):
    num_edges, num_nodes = h.shape
    base2d = (l_v * b).reshape(1, num_nodes)
    sign2d = (1.0 - 2.0 * s_c.astype(jnp.float32)).reshape(num_edges, 1)
    c = jnp.where(h == 1, w, -1.0).astype(jnp.float32)

    mu0 = jnp.zeros((num_edges, num_nodes), jnp.float32)
    tot0 = jnp.zeros((1, num_nodes), jnp.float32)

    def body(_, state):
        mu, tot = state
        mu_new, tot_new = _bp_iteration(c, base2d, sign2d, mu, tot)
        return (mu_new, tot_new)

    _, tot = jax.lax.fori_loop(0, iterations, body, (mu0, tot0))
    mu_v = base2d[0] + tot[0]
    return 1.0 / (jnp.exp(mu_v) + 1.0)
